# baseline (speedup 1.0000x reference)
import functools

import jax
import jax.numpy as jnp
from jax import lax
from jax.experimental import pallas as pl
from jax.experimental.pallas import tpu as pltpu
from jax.experimental.pallas import tpu_sc as plsc

_B = 64
_N = 32768
_NVEC = _N // 16
_NB = 256
_HIST = _NB * 16

_SIGN_INT = -2147483648


def _lane():
    return lax.iota(jnp.int32, 16)


@functools.cache
def _get_sc_select():
    sc_mesh = plsc.VectorSubcoreMesh(core_axis_name="c", subcore_axis_name="s")
    return pl.kernel(
        _sc_select_body,
        out_type=jax.ShapeDtypeStruct((_B, _N), jnp.float32),
        mesh=sc_mesh,
        compiler_params=pltpu.CompilerParams(needs_layout_passes=False),
        scratch_types=[
            pltpu.VMEM((_N,), jnp.float32),
            pltpu.VMEM((_N,), jnp.float32),
            pltpu.VMEM((_N,), jnp.float32),
            pltpu.VMEM((_HIST,), jnp.int32),
            pltpu.VMEM((_HIST,), jnp.int32),
            pltpu.VMEM((16,), jnp.int32),
        ],
    )


def _sc_select_body(inp_hbm, kidx_hbm, out_hbm, row_v, dst_v, sur_v, hist_v,
                    cum_v, kidx_v):
    wid = lax.axis_index("s") * 2 + lax.axis_index("c")
    lane = _lane()
    ones = jnp.ones((16,), jnp.int32)
    zeros = jnp.zeros((16,), jnp.int32)

    pltpu.sync_copy(kidx_hbm.at[pl.ds(wid * 16, 16)], kidx_v)

    def _clear():
        @plsc.parallel_loop(0, _NB, unroll=8)
        def _(j):
            hist_v[pl.ds(j * 16, 16)] = zeros

    def _probe(pos):
        return jnp.sum(cum_v[pl.ds(pos * 16, 16)])

    def _search(k):
        pos = jnp.int32(0)
        for s in (128, 64, 32, 16, 8, 4, 2, 1):
            c = _probe(pos + (s - 1))
            pos = jnp.where(c <= k, pos + s, pos)
        base = jnp.where(pos > 0, _probe(jnp.maximum(pos - 1, 0)), 0)
        return pos, base

    def _pick(k):
        @plsc.parallel_loop(0, _NB, unroll=8, carry=zeros)
        def acc(j, a):
            a = a + hist_v[pl.ds(j * 16, 16)]
            cum_v[pl.ds(j * 16, 16)] = a
            return a
        del acc
        return _search(k)

    def _pick0(k):
        @plsc.parallel_loop(0, 128, unroll=8, carry=zeros)
        def accn(j, a):
            a = a + hist_v[pl.ds((255 - j) * 16, 16)]
            cum_v[pl.ds(j * 16, 16)] = a
            return a

        @plsc.parallel_loop(0, 128, unroll=8, carry=accn)
        def accp(j, a):
            a = a + hist_v[pl.ds(j * 16, 16)]
            cum_v[pl.ds((128 + j) * 16, 16)] = a
            return a
        del accp
        return _search(k)

    def _scalar(v):
        return jnp.sum(jnp.where(lane == 0, v, 0))

    for r in range(2):
        row = wid * 2 + r
        pltpu.sync_copy(inp_hbm.at[row], row_v)
        k = jnp.sum(jnp.where(lane == r, kidx_v[...], 0))

        _clear()

        @plsc.parallel_loop(0, _NVEC, unroll=16)
        def p0(i):
            u = plsc.bitcast(row_v[pl.ds(i * 16, 16)], jnp.int32)
            d = lax.shift_right_logical(u, 24)
            plsc.addupdate_scatter(hist_v, [d * 16 + lane], ones)

        b0, base0 = _pick0(k)
        k1 = k - base0
        rb0 = jnp.where(b0 < 128, 255 - b0, b0 - 128)
        xm = jnp.where(rb0 >= 128, jnp.int32(-1), jnp.int32(_SIGN_INT))

        @plsc.parallel_loop(0, _NVEC // 4, unroll=4,
                            carry=jnp.full((16,), -1, jnp.int32))
        def p1a(i, wv):
            for t in range(4):
                off = (i * 4 + t) * 16
                uf = row_v[pl.ds(off, 16)]
                u = plsc.bitcast(uf, jnp.int32)
                m = lax.shift_right_logical(u, 24) == rb0
                ranks = plsc.cumsum(m.astype(jnp.int32))
                plsc.store_scatter(dst_v, [wv + ranks], uf, mask=m)
                wv = wv + plsc.all_reduce_population_count(m)
            return wv
        n1 = _scalar(p1a) + 1
        n1v = lax.div(n1 + 15, jnp.int32(16))

        _clear()

        def p1b(i, _):
            off = i * 16
            u = plsc.bitcast(dst_v[pl.ds(off, 16)], jnp.int32)
            uk = u ^ xm
            dst_v[pl.ds(off, 16)] = plsc.bitcast(uk, jnp.float32)
            msk = (off + lane) < n1
            d = lax.shift_right_logical(uk, 16) & 0xFF
            plsc.addupdate_scatter(hist_v, [d * 16 + lane], ones, mask=msk)
            return 0
        lax.fori_loop(0, n1v, p1b, 0)
        b1, base1 = _pick(k1)
        k2 = k1 - base1

        _clear()

        def p2(i, wv):
            off = i * 16
            ukf = dst_v[pl.ds(off, 16)]
            uk = plsc.bitcast(ukf, jnp.int32)
            m = ((off + lane) < n1) & (
                (lax.shift_right_logical(uk, 16) & 0xFF) == b1)
            ranks = plsc.cumsum(m.astype(jnp.int32))
            plsc.store_scatter(sur_v, [wv + ranks], ukf, mask=m)
            d = lax.shift_right_logical(uk, 8) & 0xFF
            plsc.addupdate_scatter(hist_v, [d * 16 + lane], ones, mask=m)
            return wv + plsc.all_reduce_population_count(m)
        wv2 = lax.fori_loop(0, n1v, p2, jnp.full((16,), -1, jnp.int32))
        n2 = _scalar(wv2) + 1
        b2, base2 = _pick(k2)
        k3 = k2 - base2

        _clear()

        def p3(i, _):
            off = i * 16
            uk = plsc.bitcast(sur_v[pl.ds(off, 16)], jnp.int32)
            m = ((off + lane) < n2) & (
                (lax.shift_right_logical(uk, 8) & 0xFF) == b2)
            plsc.addupdate_scatter(hist_v, [(uk & 0xFF) * 16 + lane], ones,
                                   mask=m)
            return 0
        lax.fori_loop(0, lax.div(n2 + 15, jnp.int32(16)), p3, 0)
        b3, _unused = _pick(k3)

        uu = (b0 << 24) | (b1 << 16) | (b2 << 8) | b3
        uv = jnp.full((16,), uu, jnp.int32)
        kb = jnp.where(uv < 0, uv ^ jnp.int32(_SIGN_INT), ~uv)
        tvec = plsc.bitcast(kb, jnp.float32)

        @plsc.parallel_loop(0, _NVEC, unroll=16)
        def pm(i):
            x = row_v[pl.ds(i * 16, 16)]
            row_v[pl.ds(i * 16, 16)] = jnp.where(x < tvec, jnp.float32(0.0),
                                                 x)

        pltpu.sync_copy(row_v, out_hbm.at[row])


def kernel(inputs, probs, training):
    n = inputs.shape[-1]
    kidx = jnp.maximum(
        jnp.ceil(jnp.float32(n) * probs).astype(jnp.int32) - 1, 0)
    kidx = jnp.where(training != 0, kidx, 0)
    kidx_tiles = jnp.zeros((32, 16), jnp.int32).at[:, :2].set(
        kidx.reshape(32, 2)).reshape(512)

    return _get_sc_select()(inputs, kidx_tiles)


# speedup vs baseline: 1.0525x; 1.0525x over previous
import functools

import jax
import jax.numpy as jnp
from jax import lax
from jax.experimental import pallas as pl
from jax.experimental.pallas import tpu as pltpu
from jax.experimental.pallas import tpu_sc as plsc

_B = 64
_N = 32768
_NVEC = _N // 16
_NB = 256
_HIST = _NB * 16

_SIGN_INT = -2147483648


def _lane():
    return lax.iota(jnp.int32, 16)


@functools.cache
def _get_sc_select():
    sc_mesh = plsc.VectorSubcoreMesh(core_axis_name="c", subcore_axis_name="s")
    return pl.kernel(
        _sc_select_body,
        out_type=jax.ShapeDtypeStruct((_B, _N), jnp.float32),
        mesh=sc_mesh,
        compiler_params=pltpu.CompilerParams(needs_layout_passes=False),
        scratch_types=[
            pltpu.VMEM((_N,), jnp.float32),
            pltpu.VMEM((_N,), jnp.float32),
            pltpu.VMEM((_N,), jnp.float32),
            pltpu.VMEM((_HIST,), jnp.int32),
            pltpu.VMEM((_HIST,), jnp.int32),
            pltpu.VMEM((_HIST,), jnp.int32),
            pltpu.VMEM((16,), jnp.int32),
        ],
    )


def _sc_select_body(inp_hbm, kidx_hbm, out_hbm, row_v, dst_v, sur_v, hist_v,
                    hist2_v, cum_v, kidx_v):
    wid = lax.axis_index("s") * 2 + lax.axis_index("c")
    lane = _lane()
    ones = jnp.ones((16,), jnp.int32)
    zeros = jnp.zeros((16,), jnp.int32)

    pltpu.sync_copy(kidx_hbm.at[pl.ds(wid * 16, 16)], kidx_v)

    def _clear():
        @plsc.parallel_loop(0, _NB, unroll=8)
        def _(j):
            hist_v[pl.ds(j * 16, 16)] = zeros

    def _clear2():
        @plsc.parallel_loop(0, _NB, unroll=8)
        def _(j):
            hist_v[pl.ds(j * 16, 16)] = zeros
            hist2_v[pl.ds(j * 16, 16)] = zeros

    def _probe(pos):
        return jnp.sum(cum_v[pl.ds(pos * 16, 16)])

    def _search(k):
        pos = jnp.int32(0)
        for s in (128, 64, 32, 16, 8, 4, 2, 1):
            c = _probe(pos + (s - 1))
            pos = jnp.where(c <= k, pos + s, pos)
        base = jnp.where(pos > 0, _probe(jnp.maximum(pos - 1, 0)), 0)
        return pos, base

    def _pick(k):
        @plsc.parallel_loop(0, _NB, unroll=8, carry=zeros)
        def acc(j, a):
            a = a + hist_v[pl.ds(j * 16, 16)]
            cum_v[pl.ds(j * 16, 16)] = a
            return a
        del acc
        return _search(k)

    def _pick0(k):
        @plsc.parallel_loop(0, 128, unroll=8, carry=zeros)
        def accn(j, a):
            a = a + (hist_v[pl.ds((255 - j) * 16, 16)]
                     + hist2_v[pl.ds((255 - j) * 16, 16)])
            cum_v[pl.ds(j * 16, 16)] = a
            return a

        @plsc.parallel_loop(0, 128, unroll=8, carry=accn)
        def accp(j, a):
            a = a + (hist_v[pl.ds(j * 16, 16)]
                     + hist2_v[pl.ds(j * 16, 16)])
            cum_v[pl.ds((128 + j) * 16, 16)] = a
            return a
        del accp
        return _search(k)

    def _scalar(v):
        return jnp.sum(jnp.where(lane == 0, v, 0))

    for r in range(2):
        row = wid * 2 + r
        pltpu.sync_copy(inp_hbm.at[row], row_v)
        k = jnp.sum(jnp.where(lane == r, kidx_v[...], 0))

        _clear2()

        @plsc.parallel_loop(0, _NVEC // 2, unroll=4)
        def p0(i):
            for t, hv in ((0, hist_v), (1, hist2_v)):
                u = plsc.bitcast(row_v[pl.ds((i * 2 + t) * 16, 16)],
                                 jnp.int32)
                d = lax.shift_right_logical(u, 24)
                plsc.addupdate_scatter(hv, [d * 16 + lane], ones)

        b0, base0 = _pick0(k)
        k1 = k - base0
        rb0 = jnp.where(b0 < 128, 255 - b0, b0 - 128)
        xm = jnp.where(rb0 >= 128, jnp.int32(-1), jnp.int32(_SIGN_INT))

        @plsc.parallel_loop(0, _NVEC // 4, unroll=2,
                            carry=jnp.full((16,), -1, jnp.int32))
        def p1a(i, wv):
            for t in range(4):
                off = (i * 4 + t) * 16
                uf = row_v[pl.ds(off, 16)]
                u = plsc.bitcast(uf, jnp.int32)
                m = lax.shift_right_logical(u, 24) == rb0
                ranks = plsc.cumsum(m.astype(jnp.int32))
                plsc.store_scatter(dst_v, [wv + ranks], uf, mask=m)
                wv = wv + plsc.all_reduce_population_count(m)
            return wv
        n1 = _scalar(p1a) + 1
        n1v = lax.div(n1 + 15, jnp.int32(16))

        _clear()

        def p1b(i, _):
            off = i * 16
            u = plsc.bitcast(dst_v[pl.ds(off, 16)], jnp.int32)
            uk = u ^ xm
            dst_v[pl.ds(off, 16)] = plsc.bitcast(uk, jnp.float32)
            msk = (off + lane) < n1
            d = lax.shift_right_logical(uk, 16) & 0xFF
            plsc.addupdate_scatter(hist_v, [d * 16 + lane], ones, mask=msk)
            return 0
        lax.fori_loop(0, n1v, p1b, 0)
        b1, base1 = _pick(k1)
        k2 = k1 - base1

        _clear()

        def p2(i, wv):
            off = i * 16
            ukf = dst_v[pl.ds(off, 16)]
            uk = plsc.bitcast(ukf, jnp.int32)
            m = ((off + lane) < n1) & (
                (lax.shift_right_logical(uk, 16) & 0xFF) == b1)
            ranks = plsc.cumsum(m.astype(jnp.int32))
            plsc.store_scatter(sur_v, [wv + ranks], ukf, mask=m)
            d = lax.shift_right_logical(uk, 8) & 0xFF
            plsc.addupdate_scatter(hist_v, [d * 16 + lane], ones, mask=m)
            return wv + plsc.all_reduce_population_count(m)
        wv2 = lax.fori_loop(0, n1v, p2, jnp.full((16,), -1, jnp.int32))
        n2 = _scalar(wv2) + 1
        b2, base2 = _pick(k2)
        k3 = k2 - base2

        _clear()

        def p3(i, _):
            off = i * 16
            uk = plsc.bitcast(sur_v[pl.ds(off, 16)], jnp.int32)
            m = ((off + lane) < n2) & (
                (lax.shift_right_logical(uk, 8) & 0xFF) == b2)
            plsc.addupdate_scatter(hist_v, [(uk & 0xFF) * 16 + lane], ones,
                                   mask=m)
            return 0
        lax.fori_loop(0, lax.div(n2 + 15, jnp.int32(16)), p3, 0)
        b3, _unused = _pick(k3)

        uu = (b0 << 24) | (b1 << 16) | (b2 << 8) | b3
        uv = jnp.full((16,), uu, jnp.int32)
        kb = jnp.where(uv < 0, uv ^ jnp.int32(_SIGN_INT), ~uv)
        tvec = plsc.bitcast(kb, jnp.float32)

        @plsc.parallel_loop(0, _NVEC, unroll=8)
        def pm(i):
            x = row_v[pl.ds(i * 16, 16)]
            row_v[pl.ds(i * 16, 16)] = jnp.where(x < tvec, jnp.float32(0.0),
                                                 x)

        pltpu.sync_copy(row_v, out_hbm.at[row])


def kernel(inputs, probs, training):
    n = inputs.shape[-1]
    kidx = jnp.maximum(
        jnp.ceil(jnp.float32(n) * probs).astype(jnp.int32) - 1, 0)
    kidx = jnp.where(training != 0, kidx, 0)
    kidx_tiles = jnp.zeros((32, 16), jnp.int32).at[:, :2].set(
        kidx.reshape(32, 2)).reshape(512)

    return _get_sc_select()(inputs, kidx_tiles)


# speedup vs baseline: 1.3271x; 1.2609x over previous
import functools

import jax
import jax.numpy as jnp
from jax import lax
from jax.experimental import pallas as pl
from jax.experimental.pallas import tpu as pltpu
from jax.experimental.pallas import tpu_sc as plsc

_B = 64
_N = 32768
_NVEC = _N // 16
_NB = 256
_HIST = _NB * 16

_SIGN_INT = -2147483648


def _lane():
    return lax.iota(jnp.int32, 16)


@functools.cache
def _get_sc_select():
    sc_mesh = plsc.VectorSubcoreMesh(core_axis_name="c", subcore_axis_name="s")
    return pl.kernel(
        _sc_select_body,
        out_type=jax.ShapeDtypeStruct((_B, _N), jnp.float32),
        mesh=sc_mesh,
        compiler_params=pltpu.CompilerParams(needs_layout_passes=False),
        scratch_types=[
            pltpu.VMEM((_N,), jnp.float32),
            pltpu.VMEM((_N,), jnp.float32),
            pltpu.VMEM((_N,), jnp.float32),
            pltpu.VMEM((_HIST,), jnp.int32),
            pltpu.VMEM((_HIST,), jnp.int32),
            pltpu.VMEM((16,), jnp.int32),
        ],
    )


def _sc_select_body(inp_hbm, kidx_hbm, out_hbm, row_v, dst_v, sur_v, hist_v,
                    cum_v, kidx_v):
    wid = lax.axis_index("s") * 2 + lax.axis_index("c")
    lane = _lane()
    ones = jnp.ones((16,), jnp.int32)
    zeros = jnp.zeros((16,), jnp.int32)

    pltpu.sync_copy(kidx_hbm.at[pl.ds(wid * 16, 16)], kidx_v)

    def _clear():
        @plsc.parallel_loop(0, _NB, unroll=8)
        def _(j):
            hist_v[pl.ds(j * 16, 16)] = zeros

    def _probe(pos):
        return jnp.sum(cum_v[pl.ds(pos * 16, 16)])

    def _search(k):
        pos = jnp.int32(0)
        for s in (128, 64, 32, 16, 8, 4, 2, 1):
            c = _probe(pos + (s - 1))
            pos = jnp.where(c <= k, pos + s, pos)
        base = jnp.where(pos > 0, _probe(jnp.maximum(pos - 1, 0)), 0)
        return pos, base

    def _pick(k):
        @plsc.parallel_loop(0, _NB, unroll=8, carry=zeros)
        def acc(j, a):
            a = a + hist_v[pl.ds(j * 16, 16)]
            cum_v[pl.ds(j * 16, 16)] = a
            return a
        del acc
        return _search(k)

    def _pick0(k):
        @plsc.parallel_loop(0, 128, unroll=8, carry=zeros)
        def accn(j, a):
            a = a + hist_v[pl.ds((255 - j) * 16, 16)]
            cum_v[pl.ds(j * 16, 16)] = a
            return a

        @plsc.parallel_loop(0, 128, unroll=8, carry=accn)
        def accp(j, a):
            a = a + hist_v[pl.ds(j * 16, 16)]
            cum_v[pl.ds((128 + j) * 16, 16)] = a
            return a
        del accp
        return _search(k)

    def _scalar(v):
        return jnp.sum(jnp.where(lane == 0, v, 0))

    for r in range(2):
        row = wid * 2 + r
        pltpu.sync_copy(inp_hbm.at[row], row_v)
        k = jnp.sum(jnp.where(lane == r, kidx_v[...], 0))

        _clear()

        @plsc.parallel_loop(0, _NVEC, unroll=8)
        def p0(i):
            u = plsc.bitcast(row_v[pl.ds(i * 16, 16)], jnp.int32)
            d = lax.shift_right_logical(u, 24)
            plsc.addupdate_scatter(hist_v, [d * 16 + lane], ones)

        b0, base0 = _pick0(k)
        k1 = k - base0
        rb0 = jnp.where(b0 < 128, 255 - b0, b0 - 128)
        xm = jnp.where(rb0 >= 128, jnp.int32(-1), jnp.int32(_SIGN_INT))

        _clear()

        @plsc.parallel_loop(0, _NVEC // 4, unroll=2,
                            carry=jnp.full((16,), -1, jnp.int32))
        def p1(i, wv):
            for t in range(4):
                off = (i * 4 + t) * 16
                u = plsc.bitcast(row_v[pl.ds(off, 16)], jnp.int32)
                m = lax.shift_right_logical(u, 24) == rb0
                uk = u ^ xm
                ranks = plsc.cumsum(m.astype(jnp.int32))
                plsc.store_scatter(dst_v, [wv + ranks],
                                   plsc.bitcast(uk, jnp.float32), mask=m)
                d = lax.shift_right_logical(uk, 16) & 0xFF
                plsc.addupdate_scatter(hist_v, [d * 16 + lane], ones, mask=m)
                wv = wv + plsc.all_reduce_population_count(m)
            return wv
        n1 = _scalar(p1) + 1
        n1v = lax.div(n1 + 15, jnp.int32(16))
        b1, base1 = _pick(k1)
        k2 = k1 - base1

        _clear()

        @plsc.parallel_loop(0, n1v, unroll=2,
                            carry=jnp.full((16,), -1, jnp.int32))
        def p2(i, wv):
            off = i * 16
            ukf = dst_v[pl.ds(off, 16)]
            uk = plsc.bitcast(ukf, jnp.int32)
            m = ((off + lane) < n1) & (
                (lax.shift_right_logical(uk, 16) & 0xFF) == b1)
            ranks = plsc.cumsum(m.astype(jnp.int32))
            plsc.store_scatter(sur_v, [wv + ranks], ukf, mask=m)
            d = lax.shift_right_logical(uk, 8) & 0xFF
            plsc.addupdate_scatter(hist_v, [d * 16 + lane], ones, mask=m)
            return wv + plsc.all_reduce_population_count(m)
        n2 = _scalar(p2) + 1
        b2, base2 = _pick(k2)
        k3 = k2 - base2

        _clear()

        @plsc.parallel_loop(0, lax.div(n2 + 15, jnp.int32(16)), unroll=2)
        def p3(i):
            off = i * 16
            uk = plsc.bitcast(sur_v[pl.ds(off, 16)], jnp.int32)
            m = ((off + lane) < n2) & (
                (lax.shift_right_logical(uk, 8) & 0xFF) == b2)
            plsc.addupdate_scatter(hist_v, [(uk & 0xFF) * 16 + lane], ones,
                                   mask=m)
        b3, _unused = _pick(k3)

        uu = (b0 << 24) | (b1 << 16) | (b2 << 8) | b3
        uv = jnp.full((16,), uu, jnp.int32)
        kb = jnp.where(uv < 0, uv ^ jnp.int32(_SIGN_INT), ~uv)
        tvec = plsc.bitcast(kb, jnp.float32)

        @plsc.parallel_loop(0, _NVEC, unroll=8)
        def pm(i):
            x = row_v[pl.ds(i * 16, 16)]
            row_v[pl.ds(i * 16, 16)] = jnp.where(x < tvec, jnp.float32(0.0),
                                                 x)

        pltpu.sync_copy(row_v, out_hbm.at[row])


def kernel(inputs, probs, training):
    n = inputs.shape[-1]
    kidx = jnp.maximum(
        jnp.ceil(jnp.float32(n) * probs).astype(jnp.int32) - 1, 0)
    kidx = jnp.where(training != 0, kidx, 0)
    kidx_tiles = jnp.zeros((32, 16), jnp.int32).at[:, :2].set(
        kidx.reshape(32, 2)).reshape(512)

    return _get_sc_select()(inputs, kidx_tiles)
